# Initial kernel scaffold; baseline (speedup 1.0000x reference)
import jax
import jax.numpy as jnp
from jax.experimental import pallas as pl


def kernel(mesh_node, edge_index, material_params, W1, b1, Wmu, bmu, Wlv, blv):
    raise NotImplementedError("write your pallas kernel here")


# speedup vs baseline: 7.4587x; 7.4587x over previous
import functools

import jax
import jax.numpy as jnp
from jax import lax
from jax.experimental import pallas as pl
from jax.experimental.pallas import tpu as pltpu
from jax.experimental.pallas import tpu_sc as plsc

N = 10000
E = 320000
D = 128
M = 16
H = 256
L = 128

NP = 10240
NP2 = 2 * NP
CH = 128
NTILE = 16
NCORE = 2
RPT = NP // NTILE
EP = 327680
C1 = EP // (NCORE * NTILE * CH)
C2 = EP // (NTILE * CH)
SEG = 40

_MESH = dict(core_axis_name="c", subcore_axis_name="s")


def _fill(ref, value, rows, cols):
    vec = jnp.full((16,), value, ref.dtype)

    def body(r, carry):
        for k in range(cols // 16):
            ref[r, pl.ds(k * 16, 16)] = vec
        return carry

    lax.fori_loop(0, rows, body, 0)


@functools.partial(
    pl.kernel,
    out_type=jax.ShapeDtypeStruct((NP2, L), jnp.float32),
    mesh=plsc.VectorSubcoreMesh(**_MESH),
    scratch_types=[
        pltpu.VMEM((C1, CH), jnp.int32),
        pltpu.VMEM((CH, L), jnp.float32),
        pltpu.VMEM_SHARED((NP, L), jnp.float32),
    ],
)
def _sc_degree(dst_hbm, out_hbm, idx_d, ones_v, acc):
    c = lax.axis_index("c")
    s = lax.axis_index("s")
    w = c * NTILE + s
    _fill(ones_v, 0.0, CH, L)
    for k in range(RPT // CH):
        pltpu.sync_copy(ones_v, acc.at[pl.ds(s * RPT + k * CH, CH)])
    _fill(ones_v, 1.0, CH, L)
    plsc.subcore_barrier()
    pltpu.sync_copy(dst_hbm.at[pl.ds(w * C1, C1)], idx_d)

    def step(j, carry):
        pltpu.sync_copy(ones_v, acc.at[idx_d.at[j]], add=True)
        return carry

    lax.fori_loop(0, C1, step, 0)
    plsc.subcore_barrier()
    pltpu.sync_copy(acc.at[pl.ds(s * RPT, RPT)],
                    out_hbm.at[pl.ds(c * NP + s * RPT, RPT)])


@functools.partial(
    pl.kernel,
    out_type=jax.ShapeDtypeStruct((NP2, L), jnp.float32),
    mesh=plsc.VectorSubcoreMesh(**_MESH),
    scratch_types=[
        pltpu.VMEM((SEG, CH), jnp.int32),
        pltpu.VMEM((SEG, CH), jnp.int32),
        pltpu.VMEM((CH, L), jnp.float32),
        pltpu.VMEM_SHARED((NP, L), jnp.float32),
        pltpu.SemaphoreType.DMA,
    ],
)
def _sc_agg(y_hbm, src_hbm, dst_hbm, out_hbm, idx_s, idx_d, rows_v, acc, sem):
    c = lax.axis_index("c")
    s = lax.axis_index("s")
    _fill(rows_v, 0.0, CH, L)
    for k in range(RPT // CH):
        pltpu.sync_copy(rows_v, acc.at[pl.ds(s * RPT + k * CH, CH)])
    plsc.subcore_barrier()

    def seg_body(g, carry):
        pltpu.sync_copy(
            src_hbm.at[pl.ds(c * (EP // CH) + s * C2 + g * SEG, SEG)], idx_s)
        pltpu.sync_copy(dst_hbm.at[pl.ds(s * C2 + g * SEG, SEG)], idx_d)

        def step(j, carry2):
            pltpu.async_copy(y_hbm.at[idx_s.at[j]], rows_v, sem).wait()
            pltpu.sync_copy(rows_v, acc.at[idx_d.at[j]], add=True)
            return carry2

        lax.fori_loop(0, SEG, step, 0)
        return carry

    lax.fori_loop(0, C2 // SEG, seg_body, 0)
    plsc.subcore_barrier()
    pltpu.sync_copy(acc.at[pl.ds(s * RPT, RPT)],
                    out_hbm.at[pl.ds(c * NP + s * RPT, RPT)])


def _dinv_of(d0_ref, d1_ref):
    deg = 1.0 + d0_ref[:, 0:1] + d1_ref[:, 0:1]
    return lax.rsqrt(deg)


def _tc_prep_y(mesh_pad, mat_pad, deg2):
    BR = 256
    NB = NP // BR

    def body(mesh_ref, mat_ref, d0_ref, d1_ref, y_ref):
        f = pl.program_id(1)
        dinv = _dinv_of(d0_ref, d1_ref)

        @pl.when(f == 0)
        def _():
            y_ref[...] = mesh_ref[...] * dinv

        @pl.when(f == 1)
        def _():
            y_ref[...] = jnp.concatenate(
                [mat_ref[...] * dinv, jnp.zeros((BR, L - M), jnp.float32)],
                axis=1)

    return pl.pallas_call(
        body,
        grid=(NB, 2),
        in_specs=[
            pl.BlockSpec((BR, D), lambda b, f: (b, 0)),
            pl.BlockSpec((BR, M), lambda b, f: (b, 0)),
            pl.BlockSpec((BR, L), lambda b, f: (b, 0)),
            pl.BlockSpec((BR, L), lambda b, f: (b + NB, 0)),
        ],
        out_specs=pl.BlockSpec((BR, L), lambda b, f: (b + f * NB, 0)),
        out_shape=jax.ShapeDtypeStruct((NP2, L), jnp.float32),
    )(mesh_pad, mat_pad, deg2, deg2)


def _tc_layer1(y, agg1, deg2, W1p, b1):
    BR = 256
    NB = NP // BR

    def body(y0_ref, y1_ref, a0_ref, a1_ref, d0_ref, d1_ref, w_ref, b_ref,
             o_ref):
        dinv = _dinv_of(d0_ref, d1_ref)
        t = jnp.concatenate(
            [y0_ref[...] + a0_ref[...], y1_ref[...] + a1_ref[...]],
            axis=1) * dinv
        h = jnp.dot(t, w_ref[...], preferred_element_type=jnp.float32)
        h = jnp.maximum(h + b_ref[...], 0.0)
        o_ref[...] = h * dinv

    return pl.pallas_call(
        body,
        grid=(NB, 2),
        in_specs=[
            pl.BlockSpec((BR, L), lambda b, f: (b, 0)),
            pl.BlockSpec((BR, L), lambda b, f: (b + NB, 0)),
            pl.BlockSpec((BR, L), lambda b, f: (b, 0)),
            pl.BlockSpec((BR, L), lambda b, f: (b + NB, 0)),
            pl.BlockSpec((BR, L), lambda b, f: (b, 0)),
            pl.BlockSpec((BR, L), lambda b, f: (b + NB, 0)),
            pl.BlockSpec((H, L), lambda b, f: (0, f)),
            pl.BlockSpec((1, L), lambda b, f: (0, f)),
        ],
        out_specs=pl.BlockSpec((BR, L), lambda b, f: (b + f * NB, 0)),
        out_shape=jax.ShapeDtypeStruct((NP2, L), jnp.float32),
    )(y, y, agg1, agg1, deg2, deg2, W1p, b1.reshape(1, H))


def _tc_outputs(y1, agg2, deg2, Wmu, Wlv, bmu, blv):
    BR = 256
    NB = NP // BR

    def body(y0_ref, y1_ref, a0_ref, a1_ref, d0_ref, d1_ref,
             wm_ref, wl_ref, bm_ref, bl_ref, mu_ref, lv_ref):
        dinv = _dinv_of(d0_ref, d1_ref)
        z = jnp.concatenate(
            [y0_ref[...] + a0_ref[...], y1_ref[...] + a1_ref[...]],
            axis=1) * dinv
        mu_ref[...] = jnp.dot(z, wm_ref[...],
                              preferred_element_type=jnp.float32) + bm_ref[...]
        lv_ref[...] = jnp.dot(z, wl_ref[...],
                              preferred_element_type=jnp.float32) + bl_ref[...]

    return pl.pallas_call(
        body,
        grid=(NB,),
        in_specs=[
            pl.BlockSpec((BR, L), lambda b: (b, 0)),
            pl.BlockSpec((BR, L), lambda b: (b + NB, 0)),
            pl.BlockSpec((BR, L), lambda b: (b, 0)),
            pl.BlockSpec((BR, L), lambda b: (b + NB, 0)),
            pl.BlockSpec((BR, L), lambda b: (b, 0)),
            pl.BlockSpec((BR, L), lambda b: (b + NB, 0)),
            pl.BlockSpec((H, L), lambda b: (0, 0)),
            pl.BlockSpec((H, L), lambda b: (0, 0)),
            pl.BlockSpec((1, L), lambda b: (0, 0)),
            pl.BlockSpec((1, L), lambda b: (0, 0)),
        ],
        out_specs=[
            pl.BlockSpec((BR, L), lambda b: (b, 0)),
            pl.BlockSpec((BR, L), lambda b: (b, 0)),
        ],
        out_shape=[
            jax.ShapeDtypeStruct((NP, L), jnp.float32),
            jax.ShapeDtypeStruct((NP, L), jnp.float32),
        ],
    )(y1, y1, agg2, agg2, deg2, deg2, Wmu, Wlv,
      bmu.reshape(1, L), blv.reshape(1, L))


def kernel(mesh_node, edge_index, material_params, W1, b1, Wmu, bmu, Wlv, blv):
    src = edge_index[0]
    dst = edge_index[1]
    padv = jnp.full((EP - E,), N, jnp.int32)
    src_p = jnp.concatenate([src, padv])
    dst_p = jnp.concatenate([dst, padv])
    dst1 = dst_p.reshape(EP // CH, CH)
    srcB = jnp.concatenate([src_p, src_p + NP]).reshape(2 * EP // CH, CH)

    mesh_pad = jnp.zeros((NP, D), jnp.float32).at[:N].set(mesh_node)
    mat_pad = jnp.zeros((NP, M), jnp.float32).at[:N].set(material_params)
    W1p = jnp.zeros((H, H), jnp.float32).at[:D + M].set(W1)

    deg2 = _sc_degree(dst1)
    y = _tc_prep_y(mesh_pad, mat_pad, deg2)
    agg1 = _sc_agg(y, srcB, dst1)
    y1 = _tc_layer1(y, agg1, deg2, W1p, b1)
    agg2 = _sc_agg(y1, srcB, dst1)
    mu, lv = _tc_outputs(y1, agg2, deg2, Wmu, Wlv, bmu, blv)
    return (mu[:N], lv[:N])


# speedup vs baseline: 8.2187x; 1.1019x over previous
import functools

import jax
import jax.numpy as jnp
from jax import lax
from jax.experimental import pallas as pl
from jax.experimental.pallas import tpu as pltpu
from jax.experimental.pallas import tpu_sc as plsc

N = 10000
E = 320000
D = 128
M = 16
H = 256
L = 128

NP = 10240
NP2 = 2 * NP
CH = 128
NTILE = 16
NCORE = 2
RPT = NP // NTILE
EP = 327680
C1 = EP // (NCORE * NTILE * CH)
C2 = EP // (NTILE * CH)
SEG = 40

_MESH = dict(core_axis_name="c", subcore_axis_name="s")


def _fill(ref, value, rows, cols):
    vec = jnp.full((16,), value, ref.dtype)

    def body(r, carry):
        for k in range(cols // 16):
            ref[r, pl.ds(k * 16, 16)] = vec
        return carry

    lax.fori_loop(0, rows, body, 0)


@functools.partial(
    pl.kernel,
    out_type=jax.ShapeDtypeStruct((NP2, L), jnp.float32),
    mesh=plsc.VectorSubcoreMesh(**_MESH),
    scratch_types=[
        pltpu.VMEM((C1, CH), jnp.int32),
        pltpu.VMEM((CH, L), jnp.float32),
        pltpu.VMEM_SHARED((NP, L), jnp.float32),
    ],
)
def _sc_degree(dst_hbm, out_hbm, idx_d, ones_v, acc):
    c = lax.axis_index("c")
    s = lax.axis_index("s")
    w = c * NTILE + s
    _fill(ones_v, 0.0, CH, L)
    for k in range(RPT // CH):
        pltpu.sync_copy(ones_v, acc.at[pl.ds(s * RPT + k * CH, CH)])
    _fill(ones_v, 1.0, CH, L)
    plsc.subcore_barrier()
    pltpu.sync_copy(dst_hbm.at[pl.ds(w * C1, C1)], idx_d)

    def step(j, carry):
        pltpu.sync_copy(ones_v, acc.at[idx_d.at[j]], add=True)
        return carry

    lax.fori_loop(0, C1, step, 0)
    plsc.subcore_barrier()
    pltpu.sync_copy(acc.at[pl.ds(s * RPT, RPT)],
                    out_hbm.at[pl.ds(c * NP + s * RPT, RPT)])


@functools.partial(
    pl.kernel,
    out_type=jax.ShapeDtypeStruct((NP2, L), jnp.float32),
    mesh=plsc.VectorSubcoreMesh(**_MESH),
    scratch_types=[
        pltpu.VMEM((SEG, CH), jnp.int32),
        pltpu.VMEM((SEG, CH), jnp.int32),
        pltpu.VMEM((CH, L), jnp.float32),
        pltpu.VMEM((CH, L), jnp.float32),
        pltpu.VMEM_SHARED((NP, L), jnp.float32),
        pltpu.SemaphoreType.DMA,
        pltpu.SemaphoreType.DMA,
    ],
)
def _sc_agg(y_hbm, src_hbm, dst_hbm, out_hbm, idx_s, idx_d, buf0, buf1, acc,
            sem0, sem1):
    c = lax.axis_index("c")
    s = lax.axis_index("s")
    _fill(buf0, 0.0, CH, L)
    for k in range(RPT // CH):
        pltpu.sync_copy(buf0, acc.at[pl.ds(s * RPT + k * CH, CH)])
    plsc.subcore_barrier()

    def fire(j, buf, sem):
        pltpu.async_copy(y_hbm.at[idx_s.at[j]], buf, sem)

    def drain(buf, sem):
        pltpu.make_async_copy(y_hbm.at[idx_s.at[0]], buf, sem).wait()

    def seg_body(g, carry):
        pltpu.sync_copy(
            src_hbm.at[pl.ds(c * (EP // CH) + s * C2 + g * SEG, SEG)], idx_s)
        pltpu.sync_copy(dst_hbm.at[pl.ds(s * C2 + g * SEG, SEG)], idx_d)
        fire(0, buf0, sem0)

        def pair(i, carry2):
            j = 2 * i
            drain(buf0, sem0)
            fire(j + 1, buf1, sem1)
            pltpu.sync_copy(buf0, acc.at[idx_d.at[j]], add=True)
            drain(buf1, sem1)
            fire(lax.rem(j + 2, SEG), buf0, sem0)
            pltpu.sync_copy(buf1, acc.at[idx_d.at[j + 1]], add=True)
            return carry2

        lax.fori_loop(0, SEG // 2, pair, 0)
        drain(buf0, sem0)
        return carry

    lax.fori_loop(0, C2 // SEG, seg_body, 0)
    plsc.subcore_barrier()
    pltpu.sync_copy(acc.at[pl.ds(s * RPT, RPT)],
                    out_hbm.at[pl.ds(c * NP + s * RPT, RPT)])


def _dinv_of(d0_ref, d1_ref):
    deg = 1.0 + d0_ref[:, 0:1] + d1_ref[:, 0:1]
    return lax.rsqrt(deg)


def _tc_prep_y(mesh_pad, mat_pad, deg2):
    BR = 256
    NB = NP // BR

    def body(mesh_ref, mat_ref, d0_ref, d1_ref, y_ref):
        f = pl.program_id(1)
        dinv = _dinv_of(d0_ref, d1_ref)

        @pl.when(f == 0)
        def _():
            y_ref[...] = mesh_ref[...] * dinv

        @pl.when(f == 1)
        def _():
            y_ref[...] = jnp.concatenate(
                [mat_ref[...] * dinv, jnp.zeros((BR, L - M), jnp.float32)],
                axis=1)

    return pl.pallas_call(
        body,
        grid=(NB, 2),
        in_specs=[
            pl.BlockSpec((BR, D), lambda b, f: (b, 0)),
            pl.BlockSpec((BR, M), lambda b, f: (b, 0)),
            pl.BlockSpec((BR, L), lambda b, f: (b, 0)),
            pl.BlockSpec((BR, L), lambda b, f: (b + NB, 0)),
        ],
        out_specs=pl.BlockSpec((BR, L), lambda b, f: (b + f * NB, 0)),
        out_shape=jax.ShapeDtypeStruct((NP2, L), jnp.float32),
    )(mesh_pad, mat_pad, deg2, deg2)


def _tc_layer1(y, agg1, deg2, W1p, b1):
    BR = 256
    NB = NP // BR

    def body(y0_ref, y1_ref, a0_ref, a1_ref, d0_ref, d1_ref, w_ref, b_ref,
             o_ref):
        dinv = _dinv_of(d0_ref, d1_ref)
        t = jnp.concatenate(
            [y0_ref[...] + a0_ref[...], y1_ref[...] + a1_ref[...]],
            axis=1) * dinv
        h = jnp.dot(t, w_ref[...], preferred_element_type=jnp.float32)
        h = jnp.maximum(h + b_ref[...], 0.0)
        o_ref[...] = h * dinv

    return pl.pallas_call(
        body,
        grid=(NB, 2),
        in_specs=[
            pl.BlockSpec((BR, L), lambda b, f: (b, 0)),
            pl.BlockSpec((BR, L), lambda b, f: (b + NB, 0)),
            pl.BlockSpec((BR, L), lambda b, f: (b, 0)),
            pl.BlockSpec((BR, L), lambda b, f: (b + NB, 0)),
            pl.BlockSpec((BR, L), lambda b, f: (b, 0)),
            pl.BlockSpec((BR, L), lambda b, f: (b + NB, 0)),
            pl.BlockSpec((H, L), lambda b, f: (0, f)),
            pl.BlockSpec((1, L), lambda b, f: (0, f)),
        ],
        out_specs=pl.BlockSpec((BR, L), lambda b, f: (b + f * NB, 0)),
        out_shape=jax.ShapeDtypeStruct((NP2, L), jnp.float32),
    )(y, y, agg1, agg1, deg2, deg2, W1p, b1.reshape(1, H))


def _tc_outputs(y1, agg2, deg2, Wmu, Wlv, bmu, blv):
    BR = 256
    NB = NP // BR

    def body(y0_ref, y1_ref, a0_ref, a1_ref, d0_ref, d1_ref,
             wm_ref, wl_ref, bm_ref, bl_ref, mu_ref, lv_ref):
        dinv = _dinv_of(d0_ref, d1_ref)
        z = jnp.concatenate(
            [y0_ref[...] + a0_ref[...], y1_ref[...] + a1_ref[...]],
            axis=1) * dinv
        mu_ref[...] = jnp.dot(z, wm_ref[...],
                              preferred_element_type=jnp.float32) + bm_ref[...]
        lv_ref[...] = jnp.dot(z, wl_ref[...],
                              preferred_element_type=jnp.float32) + bl_ref[...]

    return pl.pallas_call(
        body,
        grid=(NB,),
        in_specs=[
            pl.BlockSpec((BR, L), lambda b: (b, 0)),
            pl.BlockSpec((BR, L), lambda b: (b + NB, 0)),
            pl.BlockSpec((BR, L), lambda b: (b, 0)),
            pl.BlockSpec((BR, L), lambda b: (b + NB, 0)),
            pl.BlockSpec((BR, L), lambda b: (b, 0)),
            pl.BlockSpec((BR, L), lambda b: (b + NB, 0)),
            pl.BlockSpec((H, L), lambda b: (0, 0)),
            pl.BlockSpec((H, L), lambda b: (0, 0)),
            pl.BlockSpec((1, L), lambda b: (0, 0)),
            pl.BlockSpec((1, L), lambda b: (0, 0)),
        ],
        out_specs=[
            pl.BlockSpec((BR, L), lambda b: (b, 0)),
            pl.BlockSpec((BR, L), lambda b: (b, 0)),
        ],
        out_shape=[
            jax.ShapeDtypeStruct((NP, L), jnp.float32),
            jax.ShapeDtypeStruct((NP, L), jnp.float32),
        ],
    )(y1, y1, agg2, agg2, deg2, deg2, Wmu, Wlv,
      bmu.reshape(1, L), blv.reshape(1, L))


def kernel(mesh_node, edge_index, material_params, W1, b1, Wmu, bmu, Wlv, blv):
    src = edge_index[0]
    dst = edge_index[1]
    padv = jnp.full((EP - E,), N, jnp.int32)
    src_p = jnp.concatenate([src, padv])
    dst_p = jnp.concatenate([dst, padv])
    dst1 = dst_p.reshape(EP // CH, CH)
    srcB = jnp.concatenate([src_p, src_p + NP]).reshape(2 * EP // CH, CH)

    mesh_pad = jnp.zeros((NP, D), jnp.float32).at[:N].set(mesh_node)
    mat_pad = jnp.zeros((NP, M), jnp.float32).at[:N].set(material_params)
    W1p = jnp.zeros((H, H), jnp.float32).at[:D + M].set(W1)

    deg2 = _sc_degree(dst1)
    y = _tc_prep_y(mesh_pad, mat_pad, deg2)
    agg1 = _sc_agg(y, srcB, dst1)
    y1 = _tc_layer1(y, agg1, deg2, W1p, b1)
    agg2 = _sc_agg(y1, srcB, dst1)
    mu, lv = _tc_outputs(y1, agg2, deg2, Wmu, Wlv, bmu, blv)
    return (mu[:N], lv[:N])
